# Initial kernel scaffold; baseline (speedup 1.0000x reference)
import jax
import jax.numpy as jnp
from jax.experimental import pallas as pl


def kernel(x_complex, edge_index, hop_weights):
    raise NotImplementedError("write your pallas kernel here")


# speedup vs baseline: 13.1232x; 13.1232x over previous
import functools

import jax
import jax.numpy as jnp
from jax import lax
from jax.experimental import pallas as pl
from jax.experimental.pallas import tpu as pltpu
from jax.experimental.pallas import tpu_sc as plsc

_D = 128
_CHUNK = 128
_NC = 2
_NS = 16
_NW = _NC * _NS

_f32 = jnp.float32
_i32 = jnp.int32


def _rsqrt_newton(dg):
    dg = jnp.maximum(dg, 1.0)
    bits = lax.bitcast_convert_type(dg, _i32)
    y = lax.bitcast_convert_type(jnp.int32(0x5F3759DF) - (bits >> 1), _f32)
    for _ in range(4):
        y = y * (1.5 - 0.5 * dg * y * y)
    return y


def _bcast16(ref, i):
    return plsc.load_gather(ref, [jnp.full((16,), i, _i32)])


def _make_prep(npad, nch_t):
    rw = npad // _NW
    rt = npad // _NS
    mesh = plsc.VectorSubcoreMesh(core_axis_name="c", subcore_axis_name="s")

    @functools.partial(
        pl.kernel,
        out_type=(
            jax.ShapeDtypeStruct((npad,), _f32),
            jax.ShapeDtypeStruct((npad, _D), _f32),
        ),
        mesh=mesh,
        compiler_params=pltpu.CompilerParams(needs_layout_passes=False),
        scratch_types=[
            pltpu.VMEM_SHARED((npad,), _f32),
            pltpu.VMEM((nch_t, _CHUNK), _i32),
            pltpu.VMEM((_CHUNK,), _f32),
            pltpu.VMEM((rt,), _f32),
            pltpu.VMEM((rw,), _f32),
            pltpu.VMEM((rw, _D), _f32),
        ],
    )
    def prep(col2_hbm, x_hbm, dinv_hbm, y0_hbm, deg_sh, cidx, onesv, zb, degb, xb):
        c = lax.axis_index("c")
        s = lax.axis_index("s")
        wid = s * _NC + c

        def fill(i, _):
            zb[pl.ds(i * 16, 16)] = jnp.zeros((16,), _f32)
            return 0
        lax.fori_loop(0, rt // 16, fill, 0)

        def fill1(i, _):
            onesv[pl.ds(i * 16, 16)] = jnp.ones((16,), _f32)
            return 0
        lax.fori_loop(0, _CHUNK // 16, fill1, 0)

        pltpu.sync_copy(zb, deg_sh.at[pl.ds(s * rt, rt)])
        pltpu.sync_copy(col2_hbm.at[s], cidx)
        plsc.subcore_barrier()

        def fdeg(i, _):
            pltpu.sync_copy(onesv, deg_sh.at[cidx.at[i]], add=True)
            return 0
        lax.fori_loop(0, nch_t, fdeg, 0)
        plsc.subcore_barrier()

        r0 = wid * rw
        pltpu.sync_copy(deg_sh.at[pl.ds(r0, rw)], degb)

        def fnewton(j, _):
            degb[pl.ds(j * 16, 16)] = _rsqrt_newton(degb[pl.ds(j * 16, 16)])
            return 0
        lax.fori_loop(0, rw // 16, fnewton, 0)
        pltpu.sync_copy(degb, dinv_hbm.at[pl.ds(r0, rw)])

        pltpu.sync_copy(x_hbm.at[pl.ds(r0, rw)], xb)

        def frow(r, _):
            d16 = _bcast16(degb, r)
            for j in range(_D // 16):
                xb[r, pl.ds(j * 16, 16)] = xb[r, pl.ds(j * 16, 16)] * d16
            return 0
        lax.fori_loop(0, rw, frow, 0)
        pltpu.sync_copy(xb, y0_hbm.at[pl.ds(r0, rw)])

    return prep


def _make_hop(npad, nch_w):
    rt = npad // _NS
    zr = _CHUNK
    mesh = plsc.VectorSubcoreMesh(core_axis_name="c", subcore_axis_name="s")

    @functools.partial(
        pl.kernel,
        out_type=jax.ShapeDtypeStruct((2 * npad, _D), _f32),
        mesh=mesh,
        compiler_params=pltpu.CompilerParams(needs_layout_passes=False),
        scratch_types=[
            pltpu.VMEM_SHARED((npad, _D), _f32),
            pltpu.VMEM((nch_w, _CHUNK), _i32),
            pltpu.VMEM((nch_w, _CHUNK), _i32),
            pltpu.VMEM((_CHUNK, _D), _f32),
            pltpu.SemaphoreType.DMA,
        ],
    )
    def hop(row2_hbm, col2_hbm, y_hbm, out_hbm, acc_sh, ridx, cidx, rowsv, sem):
        c = lax.axis_index("c")
        s = lax.axis_index("s")
        wid = s * _NC + c

        def fz(i, _):
            for j in range(_D // 16):
                rowsv[i, pl.ds(j * 16, 16)] = jnp.zeros((16,), _f32)
            return 0
        lax.fori_loop(0, zr, fz, 0)

        def fza(i, _):
            pltpu.sync_copy(rowsv, acc_sh.at[pl.ds(s * rt + i * zr, zr)])
            return 0
        lax.fori_loop(0, rt // zr, fza, 0)

        pltpu.sync_copy(row2_hbm.at[wid], ridx)
        pltpu.sync_copy(col2_hbm.at[wid], cidx)
        plsc.subcore_barrier()

        def fedge(i, _):
            pltpu.async_copy(y_hbm.at[cidx.at[i]], rowsv, sem).wait()
            pltpu.sync_copy(rowsv, acc_sh.at[ridx.at[i]], add=True)
            return 0
        lax.fori_loop(0, nch_w, fedge, 0)
        plsc.subcore_barrier()

        pltpu.sync_copy(acc_sh.at[pl.ds(s * rt, rt)],
                        out_hbm.at[pl.ds(c * npad + s * rt, rt)])

    return hop


def _make_combine(npad):
    rw = npad // _NW
    mesh = plsc.VectorSubcoreMesh(core_axis_name="c", subcore_axis_name="s")

    @functools.partial(
        pl.kernel,
        out_type=(
            jax.ShapeDtypeStruct((npad, _D), _f32),
            jax.ShapeDtypeStruct((npad, _D), _f32),
        ),
        mesh=mesh,
        compiler_params=pltpu.CompilerParams(needs_layout_passes=False),
        scratch_types=[
            pltpu.VMEM((rw, _D), _f32),
            pltpu.VMEM((rw, _D), _f32),
            pltpu.VMEM((rw,), _f32),
        ],
    )
    def combine(acc_hbm, dinv_hbm, h1_hbm, y1_hbm, ab, bb, degb):
        c = lax.axis_index("c")
        s = lax.axis_index("s")
        wid = s * _NC + c
        r0 = wid * rw
        pltpu.sync_copy(acc_hbm.at[pl.ds(r0, rw)], ab)
        pltpu.sync_copy(acc_hbm.at[pl.ds(npad + r0, rw)], bb)
        pltpu.sync_copy(dinv_hbm.at[pl.ds(r0, rw)], degb)

        def frow(r, _):
            d16 = _bcast16(degb, r)
            for j in range(_D // 16):
                h = (ab[r, pl.ds(j * 16, 16)] + bb[r, pl.ds(j * 16, 16)]) * d16
                ab[r, pl.ds(j * 16, 16)] = h
                bb[r, pl.ds(j * 16, 16)] = h * d16
            return 0
        lax.fori_loop(0, rw, frow, 0)
        pltpu.sync_copy(ab, h1_hbm.at[pl.ds(r0, rw)])
        pltpu.sync_copy(bb, y1_hbm.at[pl.ds(r0, rw)])

    return combine


def _make_final(npad):
    rw = npad // _NW
    mesh = plsc.VectorSubcoreMesh(core_axis_name="c", subcore_axis_name="s")

    @functools.partial(
        pl.kernel,
        out_type=jax.ShapeDtypeStruct((npad, _D), _f32),
        mesh=mesh,
        compiler_params=pltpu.CompilerParams(needs_layout_passes=False),
        scratch_types=[
            pltpu.VMEM((rw, _D), _f32),
            pltpu.VMEM((rw, _D), _f32),
            pltpu.VMEM((rw,), _f32),
            pltpu.VMEM((16,), _f32),
        ],
    )
    def final(x_hbm, h1_hbm, acc_hbm, dinv_hbm, hw_hbm, res_hbm, ab, tb, degb, wbuf):
        c = lax.axis_index("c")
        s = lax.axis_index("s")
        wid = s * _NC + c
        r0 = wid * rw

        pltpu.sync_copy(hw_hbm, wbuf)
        v = wbuf[...]
        e = jnp.exp(v - jnp.max(v))
        wv = e / jnp.sum(e)
        lane = lax.broadcasted_iota(_i32, (16,), 0)
        zero16 = jnp.zeros((16,), _f32)
        w0 = jnp.sum(jnp.where(lane == 0, wv, zero16))
        w1 = jnp.sum(jnp.where(lane == 1, wv, zero16))
        w2 = jnp.sum(jnp.where(lane == 2, wv, zero16))

        pltpu.sync_copy(acc_hbm.at[pl.ds(r0, rw)], ab)
        pltpu.sync_copy(acc_hbm.at[pl.ds(npad + r0, rw)], tb)
        pltpu.sync_copy(dinv_hbm.at[pl.ds(r0, rw)], degb)

        def f1(r, _):
            d16 = _bcast16(degb, r)
            for j in range(_D // 16):
                ab[r, pl.ds(j * 16, 16)] = (
                    (ab[r, pl.ds(j * 16, 16)] + tb[r, pl.ds(j * 16, 16)]) * d16 * w2)
            return 0
        lax.fori_loop(0, rw, f1, 0)

        pltpu.sync_copy(h1_hbm.at[pl.ds(r0, rw)], tb)

        def f2(r, _):
            for j in range(_D // 16):
                ab[r, pl.ds(j * 16, 16)] = (
                    ab[r, pl.ds(j * 16, 16)] + w1 * tb[r, pl.ds(j * 16, 16)])
            return 0
        lax.fori_loop(0, rw, f2, 0)

        pltpu.sync_copy(x_hbm.at[pl.ds(r0, rw)], tb)

        def f3(r, _):
            for j in range(_D // 16):
                ab[r, pl.ds(j * 16, 16)] = (
                    ab[r, pl.ds(j * 16, 16)] + w0 * tb[r, pl.ds(j * 16, 16)])
            return 0
        lax.fori_loop(0, rw, f3, 0)
        pltpu.sync_copy(ab, res_hbm.at[pl.ds(r0, rw)])

    return final


def kernel(x_complex, edge_index, hop_weights):
    n, d = x_complex.shape
    assert d == _D
    ne = edge_index.shape[1]
    ei = edge_index.astype(_i32)

    loops = jnp.arange(n, dtype=_i32)
    etot = ne + n
    epad = -(-etot // (_NW * _CHUNK)) * (_NW * _CHUNK)
    pad = jnp.full((epad - etot,), n, _i32)
    row = jnp.concatenate([ei[0], loops, pad])
    col = jnp.concatenate([ei[1], loops, pad])
    row3w = row.reshape(_NW, epad // _CHUNK // _NW, _CHUNK)
    col3w = col.reshape(_NW, epad // _CHUNK // _NW, _CHUNK)
    col3t = col.reshape(_NS, epad // _CHUNK // _NS, _CHUNK)

    npad = -(-(n + 1) // 256) * 256
    x_pad = jnp.zeros((npad, d), _f32).at[:n].set(x_complex)
    nw = hop_weights.shape[0]
    hw16 = jnp.full((16,), -1e30, _f32).at[:nw].set(hop_weights.astype(_f32))

    nch_t = epad // _CHUNK // _NS
    nch_w = epad // _CHUNK // _NW

    prep = _make_prep(npad, nch_t)
    hop = _make_hop(npad, nch_w)
    combine = _make_combine(npad)
    final = _make_final(npad)

    dinv, y0 = prep(col3t, x_pad)
    acc1 = hop(row3w, col3w, y0)
    h1, y1 = combine(acc1, dinv)
    acc2 = hop(row3w, col3w, y1)
    res = final(x_pad, h1, acc2, dinv, hw16)
    return res[:n]
